# Initial kernel scaffold; baseline (speedup 1.0000x reference)
import functools

import jax
import jax.numpy as jnp
from jax import lax
from jax.experimental import pallas as pl
from jax.experimental.pallas import tpu as pltpu
from jax.experimental.pallas import tpu_sc as plsc

N_ANCHORS = 1024
DIM = 128
K = 10
MARGIN = 0.5
NEG_FILL = -3.0
BLOCK_W = 1000
N_BLOCKS = 100000 // BLOCK_W


def _make_sc_gather():
    info = plsc.get_sparse_core_info()
    nc, ns = info.num_cores, info.num_subcores
    nw = nc * ns
    b_per_w = N_ANCHORS // nw

    mesh = plsc.VectorSubcoreMesh(core_axis_name="c", subcore_axis_name="s")

    @functools.partial(
        pl.kernel,
        mesh=mesh,
        out_type=[
            jax.ShapeDtypeStruct((N_ANCHORS, DIM), jnp.float32),
            jax.ShapeDtypeStruct((N_ANCHORS, DIM), jnp.float32),
        ],
        scratch_types=[
            pltpu.VMEM((b_per_w,), jnp.int32),
            pltpu.VMEM((b_per_w,), jnp.int32),
            pltpu.VMEM((b_per_w, DIM), jnp.float32),
            pltpu.VMEM((b_per_w, DIM), jnp.float32),
            pltpu.SemaphoreType.DMA,
            pltpu.SemaphoreType.DMA,
        ],
    )
    def sc_gather(idx1_hbm, idx2_hbm, t1_hbm, t2_hbm, o1_hbm, o2_hbm,
                  idx1_v, idx2_v, rows1_v, rows2_v, sem1, sem2):
        wid = lax.axis_index("s") * nc + lax.axis_index("c")
        base = wid * b_per_w
        pltpu.sync_copy(idx1_hbm.at[pl.ds(base, b_per_w)], idx1_v)
        pltpu.sync_copy(idx2_hbm.at[pl.ds(base, b_per_w)], idx2_v)
        cp1 = pltpu.async_copy(t1_hbm.at[idx1_v], rows1_v, sem1)
        cp2 = pltpu.async_copy(t2_hbm.at[idx2_v], rows2_v, sem2)
        cp1.wait()
        cp2.wait()
        pltpu.sync_copy(rows1_v, o1_hbm.at[pl.ds(base, b_per_w)])
        pltpu.sync_copy(rows2_v, o2_hbm.at[pl.ds(base, b_per_w)])

    return sc_gather


def _tc_body(a1_ref, a2_ref, t2_ref, t1_ref, out_ref, anc_ref, top_ref,
             acc_ref):
    s = pl.program_id(0)
    j = pl.program_id(1)
    last = N_BLOCKS - 1

    @pl.when(j == 0)
    def _init():
        @pl.when(s == 0)
        def _():
            a = a1_ref[...]
            nrm = jnp.maximum(
                jnp.sqrt(jnp.sum(a * a, axis=1, keepdims=True)), 1e-12)
            anc_ref[...] = a / nrm
            acc_ref[0, 0] = 0.0

        @pl.when(s == 1)
        def _():
            a = a2_ref[...]
            nrm = jnp.maximum(
                jnp.sqrt(jnp.sum(a * a, axis=1, keepdims=True)), 1e-12)
            anc_ref[...] = a / nrm

        top_ref[...] = jnp.full((N_ANCHORS, 128), NEG_FILL, jnp.float32)

    def _merge_block(blk):
        nrm = jnp.maximum(
            jnp.sqrt(jnp.sum(blk * blk, axis=1, keepdims=True)), 1e-12)
        blkn = blk / nrm
        sims = lax.dot_general(
            anc_ref[...], blkn, (((1,), (1,)), ((), ())),
            preferred_element_type=jnp.float32)
        run = top_ref[...]
        for k in range(K):
            m = jnp.maximum(jnp.max(sims, axis=1, keepdims=True),
                            jnp.max(run, axis=1, keepdims=True))
            sims = jnp.where(sims == m, NEG_FILL, sims)
            run = jnp.where(run == m, NEG_FILL, run)
            top_ref[:, k:k + 1] = m

    @pl.when(s == 0)
    def _():
        _merge_block(t2_ref[...])

    @pl.when(s == 1)
    def _():
        _merge_block(t1_ref[...])

    @pl.when(j == last)
    def _side_loss():
        a1 = a1_ref[...]
        a2 = a2_ref[...]
        num = jnp.sum(a1 * a2, axis=1, keepdims=True)
        den = (jnp.sqrt(jnp.sum(a1 * a1, axis=1, keepdims=True)) *
               jnp.sqrt(jnp.sum(a2 * a2, axis=1, keepdims=True)))
        d_m1 = (1.0 + MARGIN - num / den) - 1.0
        terms = jnp.maximum(d_m1 + top_ref[...], 0.0)
        acc_ref[0, 0] += jnp.sum(terms)

        @pl.when(s == 1)
        def _emit():
            out_ref[0, 0] = acc_ref[0, 0] / (N_ANCHORS * K)


def _tc_loss(a1, a2, out1, out2):
    return pl.pallas_call(
        _tc_body,
        grid=(2, N_BLOCKS),
        in_specs=[
            pl.BlockSpec((N_ANCHORS, DIM), lambda s, j: (0, 0)),
            pl.BlockSpec((N_ANCHORS, DIM), lambda s, j: (0, 0)),
            pl.BlockSpec((BLOCK_W, DIM), lambda s, j: (j, 0)),
            pl.BlockSpec((BLOCK_W, DIM), lambda s, j: (j, 0)),
        ],
        out_specs=pl.BlockSpec((1, 1), lambda s, j: (0, 0)),
        out_shape=jax.ShapeDtypeStruct((1, 1), jnp.float32),
        scratch_shapes=[
            pltpu.VMEM((N_ANCHORS, DIM), jnp.float32),
            pltpu.VMEM((N_ANCHORS, 128), jnp.float32),
            pltpu.SMEM((1, 1), jnp.float32),
        ],
    )(a1, a2, out2, out1)


def kernel(out1, out2, anchor_links):
    anchor1 = anchor_links[:, 0].astype(jnp.int32)
    anchor2 = anchor_links[:, 1].astype(jnp.int32)
    a1, a2 = _sc_gather(anchor1, anchor2, out1, out2)
    loss = _tc_loss(a1, a2, out1, out2)
    return loss[0, 0]


# speedup vs baseline: 118.5798x; 118.5798x over previous
import functools

import jax
import jax.numpy as jnp
from jax import lax
from jax.experimental import pallas as pl
from jax.experimental.pallas import tpu as pltpu
from jax.experimental.pallas import tpu_sc as plsc

N_ANCHORS = 1024
DIM = 128
K = 10
MARGIN = 0.5
NEG_FILL = -3.0
BLOCK_W = 1000
N_BLOCKS = 100000 // BLOCK_W


def _make_sc_gather():
    info = plsc.get_sparse_core_info()
    nc, ns = info.num_cores, info.num_subcores
    nw = nc * ns
    b_per_w = N_ANCHORS // nw

    mesh = plsc.VectorSubcoreMesh(core_axis_name="c", subcore_axis_name="s")

    @functools.partial(
        pl.kernel,
        mesh=mesh,
        out_type=[
            jax.ShapeDtypeStruct((N_ANCHORS, DIM), jnp.float32),
            jax.ShapeDtypeStruct((N_ANCHORS, DIM), jnp.float32),
        ],
        scratch_types=[
            pltpu.VMEM((b_per_w,), jnp.int32),
            pltpu.VMEM((b_per_w,), jnp.int32),
            pltpu.VMEM((b_per_w, DIM), jnp.float32),
            pltpu.VMEM((b_per_w, DIM), jnp.float32),
            pltpu.SemaphoreType.DMA,
            pltpu.SemaphoreType.DMA,
        ],
    )
    def sc_gather(idx1_hbm, idx2_hbm, t1_hbm, t2_hbm, o1_hbm, o2_hbm,
                  idx1_v, idx2_v, rows1_v, rows2_v, sem1, sem2):
        wid = lax.axis_index("s") * nc + lax.axis_index("c")
        base = wid * b_per_w
        pltpu.sync_copy(idx1_hbm.at[pl.ds(base, b_per_w)], idx1_v)
        pltpu.sync_copy(idx2_hbm.at[pl.ds(base, b_per_w)], idx2_v)
        cp1 = pltpu.async_copy(t1_hbm.at[idx1_v], rows1_v, sem1)
        cp2 = pltpu.async_copy(t2_hbm.at[idx2_v], rows2_v, sem2)
        cp1.wait()
        cp2.wait()
        pltpu.sync_copy(rows1_v, o1_hbm.at[pl.ds(base, b_per_w)])
        pltpu.sync_copy(rows2_v, o2_hbm.at[pl.ds(base, b_per_w)])

    return sc_gather


_sc_gather_cache = []


def _sc_gather(anchor1, anchor2, out1, out2):
    if not _sc_gather_cache:
        _sc_gather_cache.append(_make_sc_gather())
    return _sc_gather_cache[0](anchor1, anchor2, out1, out2)


def _tc_body(a1_ref, a2_ref, t2_ref, t1_ref, out_ref, anc_ref, top_ref,
             acc_ref):
    s = pl.program_id(0)
    j = pl.program_id(1)
    last = N_BLOCKS - 1

    @pl.when(j == 0)
    def _init():
        @pl.when(s == 0)
        def _():
            a = a1_ref[...]
            nrm = jnp.maximum(
                jnp.sqrt(jnp.sum(a * a, axis=1, keepdims=True)), 1e-12)
            anc_ref[...] = a / nrm
            acc_ref[0, 0] = 0.0

        @pl.when(s == 1)
        def _():
            a = a2_ref[...]
            nrm = jnp.maximum(
                jnp.sqrt(jnp.sum(a * a, axis=1, keepdims=True)), 1e-12)
            anc_ref[...] = a / nrm

        top_ref[...] = jnp.full((N_ANCHORS, 128), NEG_FILL, jnp.float32)

    def _merge_block(blk):
        nrm = jnp.maximum(
            jnp.sqrt(jnp.sum(blk * blk, axis=1, keepdims=True)), 1e-12)
        blkn = blk / nrm
        sims = lax.dot_general(
            anc_ref[...], blkn, (((1,), (1,)), ((), ())),
            preferred_element_type=jnp.float32)
        run = top_ref[...]
        for k in range(K):
            m = jnp.maximum(jnp.max(sims, axis=1, keepdims=True),
                            jnp.max(run, axis=1, keepdims=True))
            sims = jnp.where(sims == m, NEG_FILL, sims)
            run = jnp.where(run == m, NEG_FILL, run)
            top_ref[:, k:k + 1] = m

    @pl.when(s == 0)
    def _():
        _merge_block(t2_ref[...])

    @pl.when(s == 1)
    def _():
        _merge_block(t1_ref[...])

    @pl.when(j == last)
    def _side_loss():
        a1 = a1_ref[...]
        a2 = a2_ref[...]
        num = jnp.sum(a1 * a2, axis=1, keepdims=True)
        den = (jnp.sqrt(jnp.sum(a1 * a1, axis=1, keepdims=True)) *
               jnp.sqrt(jnp.sum(a2 * a2, axis=1, keepdims=True)))
        d_m1 = (1.0 + MARGIN - num / den) - 1.0
        terms = jnp.maximum(d_m1 + top_ref[...], 0.0)
        acc_ref[0, 0] += jnp.sum(terms)

        @pl.when(s == 1)
        def _emit():
            out_ref[...] = jnp.broadcast_to(
                acc_ref[0, 0] / (N_ANCHORS * K), (1, 1))


def _tc_loss(a1, a2, out1, out2):
    return pl.pallas_call(
        _tc_body,
        grid=(2, N_BLOCKS),
        in_specs=[
            pl.BlockSpec((N_ANCHORS, DIM), lambda s, j: (0, 0)),
            pl.BlockSpec((N_ANCHORS, DIM), lambda s, j: (0, 0)),
            pl.BlockSpec((BLOCK_W, DIM), lambda s, j: (j, 0)),
            pl.BlockSpec((BLOCK_W, DIM), lambda s, j: (j, 0)),
        ],
        out_specs=pl.BlockSpec((1, 1), lambda s, j: (0, 0)),
        out_shape=jax.ShapeDtypeStruct((1, 1), jnp.float32),
        scratch_shapes=[
            pltpu.VMEM((N_ANCHORS, DIM), jnp.float32),
            pltpu.VMEM((N_ANCHORS, 128), jnp.float32),
            pltpu.SMEM((1, 1), jnp.float32),
        ],
    )(a1, a2, out2, out1)


def kernel(out1, out2, anchor_links):
    anchor1 = anchor_links[:, 0].astype(jnp.int32)
    anchor2 = anchor_links[:, 1].astype(jnp.int32)
    a1, a2 = _sc_gather(anchor1, anchor2, out1, out2)
    loss = _tc_loss(a1, a2, out1, out2)
    return loss[0, 0]


# speedup vs baseline: 195.4938x; 1.6486x over previous
import functools

import jax
import jax.numpy as jnp
from jax import lax
from jax.experimental import pallas as pl
from jax.experimental.pallas import tpu as pltpu
from jax.experimental.pallas import tpu_sc as plsc

N_ANCHORS = 1024
DIM = 128
K = 10
MARGIN = 0.5
NEG_FILL = -3.0
BLOCK_W = 1000
N_BLOCKS = 100000 // BLOCK_W


def _make_sc_gather():
    info = plsc.get_sparse_core_info()
    nc, ns = info.num_cores, info.num_subcores
    nw = nc * ns
    b_per_w = N_ANCHORS // nw

    mesh = plsc.VectorSubcoreMesh(core_axis_name="c", subcore_axis_name="s")

    @functools.partial(
        pl.kernel,
        mesh=mesh,
        out_type=[
            jax.ShapeDtypeStruct((N_ANCHORS, DIM), jnp.float32),
            jax.ShapeDtypeStruct((N_ANCHORS, DIM), jnp.float32),
        ],
        scratch_types=[
            pltpu.VMEM((b_per_w,), jnp.int32),
            pltpu.VMEM((b_per_w,), jnp.int32),
            pltpu.VMEM((b_per_w, DIM), jnp.float32),
            pltpu.VMEM((b_per_w, DIM), jnp.float32),
            pltpu.SemaphoreType.DMA,
            pltpu.SemaphoreType.DMA,
        ],
    )
    def sc_gather(idx1_hbm, idx2_hbm, t1_hbm, t2_hbm, o1_hbm, o2_hbm,
                  idx1_v, idx2_v, rows1_v, rows2_v, sem1, sem2):
        wid = lax.axis_index("s") * nc + lax.axis_index("c")
        base = wid * b_per_w
        pltpu.sync_copy(idx1_hbm.at[pl.ds(base, b_per_w)], idx1_v)
        pltpu.sync_copy(idx2_hbm.at[pl.ds(base, b_per_w)], idx2_v)
        cp1 = pltpu.async_copy(t1_hbm.at[idx1_v], rows1_v, sem1)
        cp2 = pltpu.async_copy(t2_hbm.at[idx2_v], rows2_v, sem2)
        cp1.wait()
        cp2.wait()
        pltpu.sync_copy(rows1_v, o1_hbm.at[pl.ds(base, b_per_w)])
        pltpu.sync_copy(rows2_v, o2_hbm.at[pl.ds(base, b_per_w)])

    return sc_gather


_sc_gather_cache = []


def _sc_gather(anchor1, anchor2, out1, out2):
    if not _sc_gather_cache:
        _sc_gather_cache.append(_make_sc_gather())
    return _sc_gather_cache[0](anchor1, anchor2, out1, out2)


BIG_FILL = 1e9


def _tc_body(a1_ref, a2_ref, t2_ref, t1_ref, out_ref, anc_ref, top_ref,
             m_ref, acc_ref, flag_ref):
    s = pl.program_id(0)
    j = pl.program_id(1)
    last = N_BLOCKS - 1
    lane = lax.broadcasted_iota(jnp.int32, (N_ANCHORS, 128), 1)

    @pl.when(j == 0)
    def _init():
        @pl.when(s == 0)
        def _():
            a = a1_ref[...]
            nrm = jnp.maximum(
                jnp.sqrt(jnp.sum(a * a, axis=1, keepdims=True)), 1e-12)
            anc_ref[...] = a / nrm
            acc_ref[0, 0] = 0.0

        @pl.when(s == 1)
        def _():
            a = a2_ref[...]
            nrm = jnp.maximum(
                jnp.sqrt(jnp.sum(a * a, axis=1, keepdims=True)), 1e-12)
            anc_ref[...] = a / nrm

        sentinel = NEG_FILL - lane.astype(jnp.float32) * 1e-3
        top_ref[...] = jnp.where(lane < K, sentinel, BIG_FILL)

    def _insert(m):
        run = top_ref[...]
        mn = jnp.min(run, axis=1, keepdims=True)
        hit = (run == mn) & (m > mn)
        top_ref[...] = jnp.where(hit, jnp.broadcast_to(m, run.shape), run)
        m_ref[...] = m
        flag_ref[0, 0] = jnp.max(m - mn)

    def _merge_block(blk):
        nrm = jnp.maximum(
            jnp.sqrt(jnp.sum(blk * blk, axis=1, keepdims=True)), 1e-12)
        blkn = blk / nrm
        sims = lax.dot_general(
            anc_ref[...], blkn, (((1,), (1,)), ((), ())),
            preferred_element_type=jnp.float32)
        _insert(jnp.max(sims, axis=1, keepdims=True))
        for _ in range(1, K):
            @pl.when(flag_ref[0, 0] > 0.0)
            def _next_candidate():
                mp = m_ref[...]
                m = jnp.max(jnp.where(sims < mp, sims, NEG_FILL),
                            axis=1, keepdims=True)
                _insert(m)

    @pl.when(s == 0)
    def _():
        _merge_block(t2_ref[...])

    @pl.when(s == 1)
    def _():
        _merge_block(t1_ref[...])

    @pl.when(j == last)
    def _side_loss():
        a1 = a1_ref[...]
        a2 = a2_ref[...]
        num = jnp.sum(a1 * a2, axis=1, keepdims=True)
        den = (jnp.sqrt(jnp.sum(a1 * a1, axis=1, keepdims=True)) *
               jnp.sqrt(jnp.sum(a2 * a2, axis=1, keepdims=True)))
        d_m1 = (1.0 + MARGIN - num / den) - 1.0
        terms = jnp.where(lane < K,
                          jnp.maximum(d_m1 + top_ref[...], 0.0), 0.0)
        acc_ref[0, 0] += jnp.sum(terms)

        @pl.when(s == 1)
        def _emit():
            out_ref[...] = jnp.broadcast_to(
                acc_ref[0, 0] / (N_ANCHORS * K), (1, 1))


def _tc_loss(a1, a2, out1, out2):
    return pl.pallas_call(
        _tc_body,
        grid=(2, N_BLOCKS),
        in_specs=[
            pl.BlockSpec((N_ANCHORS, DIM), lambda s, j: (0, 0)),
            pl.BlockSpec((N_ANCHORS, DIM), lambda s, j: (0, 0)),
            pl.BlockSpec((BLOCK_W, DIM), lambda s, j: (j, 0)),
            pl.BlockSpec((BLOCK_W, DIM), lambda s, j: (j, 0)),
        ],
        out_specs=pl.BlockSpec((1, 1), lambda s, j: (0, 0)),
        out_shape=jax.ShapeDtypeStruct((1, 1), jnp.float32),
        scratch_shapes=[
            pltpu.VMEM((N_ANCHORS, DIM), jnp.float32),
            pltpu.VMEM((N_ANCHORS, 128), jnp.float32),
            pltpu.VMEM((N_ANCHORS, 1), jnp.float32),
            pltpu.SMEM((1, 1), jnp.float32),
            pltpu.SMEM((1, 1), jnp.float32),
        ],
    )(a1, a2, out2, out1)


def kernel(out1, out2, anchor_links):
    anchor1 = anchor_links[:, 0].astype(jnp.int32)
    anchor2 = anchor_links[:, 1].astype(jnp.int32)
    a1, a2 = _sc_gather(anchor1, anchor2, out1, out2)
    loss = _tc_loss(a1, a2, out1, out2)
    return loss[0, 0]


# speedup vs baseline: 502.8808x; 2.5724x over previous
import functools

import jax
import jax.numpy as jnp
from jax import lax
from jax.experimental import pallas as pl
from jax.experimental.pallas import tpu as pltpu
from jax.experimental.pallas import tpu_sc as plsc

N_ANCHORS = 1024
DIM = 128
K = 10
MARGIN = 0.5
NEG_FILL = -3.0
BLOCK_W = 1000
N_BLOCKS = 100000 // BLOCK_W


def _make_sc_gather():
    info = plsc.get_sparse_core_info()
    nc, ns = info.num_cores, info.num_subcores
    nw = nc * ns
    b_per_w = N_ANCHORS // nw

    mesh = plsc.VectorSubcoreMesh(core_axis_name="c", subcore_axis_name="s")

    @functools.partial(
        pl.kernel,
        mesh=mesh,
        out_type=[
            jax.ShapeDtypeStruct((N_ANCHORS, DIM), jnp.float32),
            jax.ShapeDtypeStruct((N_ANCHORS, DIM), jnp.float32),
        ],
        scratch_types=[
            pltpu.VMEM((b_per_w,), jnp.int32),
            pltpu.VMEM((b_per_w,), jnp.int32),
            pltpu.VMEM((b_per_w, DIM), jnp.float32),
            pltpu.VMEM((b_per_w, DIM), jnp.float32),
            pltpu.SemaphoreType.DMA,
            pltpu.SemaphoreType.DMA,
        ],
    )
    def sc_gather(idx1_hbm, idx2_hbm, t1_hbm, t2_hbm, o1_hbm, o2_hbm,
                  idx1_v, idx2_v, rows1_v, rows2_v, sem1, sem2):
        wid = lax.axis_index("s") * nc + lax.axis_index("c")
        base = wid * b_per_w
        pltpu.sync_copy(idx1_hbm.at[pl.ds(base, b_per_w)], idx1_v)
        pltpu.sync_copy(idx2_hbm.at[pl.ds(base, b_per_w)], idx2_v)
        cp1 = pltpu.async_copy(t1_hbm.at[idx1_v], rows1_v, sem1)
        cp2 = pltpu.async_copy(t2_hbm.at[idx2_v], rows2_v, sem2)
        cp1.wait()
        cp2.wait()
        pltpu.sync_copy(rows1_v, o1_hbm.at[pl.ds(base, b_per_w)])
        pltpu.sync_copy(rows2_v, o2_hbm.at[pl.ds(base, b_per_w)])

    return sc_gather


_sc_gather_cache = []


def _sc_gather(anchor1, anchor2, out1, out2):
    if not _sc_gather_cache:
        _sc_gather_cache.append(_make_sc_gather())
    return _sc_gather_cache[0](anchor1, anchor2, out1, out2)


N_TILES = (BLOCK_W + 127) // 128


def _tc_body(a1_ref, a2_ref, t2_ref, t1_ref, out_ref, anc_ref, A_ref, B_ref,
             C_ref, acc_ref):
    s = pl.program_id(0)
    j = pl.program_id(1)
    last = N_BLOCKS - 1

    @pl.when(j == 0)
    def _init():
        @pl.when(s == 0)
        def _():
            a = a1_ref[...]
            nrm = jnp.maximum(
                jnp.sqrt(jnp.sum(a * a, axis=1, keepdims=True)), 1e-12)
            anc_ref[...] = (a / nrm).astype(jnp.bfloat16)
            acc_ref[0, 0] = 0.0

        @pl.when(s == 1)
        def _():
            a = a2_ref[...]
            nrm = jnp.maximum(
                jnp.sqrt(jnp.sum(a * a, axis=1, keepdims=True)), 1e-12)
            anc_ref[...] = (a / nrm).astype(jnp.bfloat16)

        fill = jnp.full((N_ANCHORS, 128), NEG_FILL, jnp.float32)
        A_ref[...] = fill
        B_ref[...] = fill
        C_ref[...] = fill

    def _merge_block(blk):
        nrm = jnp.maximum(
            jnp.sqrt(jnp.sum(blk * blk, axis=1, keepdims=True)), 1e-12)
        blkn = (blk / nrm).astype(jnp.bfloat16)
        sims = lax.dot_general(
            anc_ref[...], blkn, (((1,), (1,)), ((), ())),
            preferred_element_type=jnp.float32)
        a = A_ref[...]
        b = B_ref[...]
        c = C_ref[...]
        for t in range(N_TILES):
            lo = t * 128
            hi = min(lo + 128, BLOCK_W)
            tile = sims[:, lo:hi]
            if hi - lo < 128:
                pad = jnp.full((N_ANCHORS, 128 - (hi - lo)), NEG_FILL,
                               jnp.float32)
                tile = jnp.concatenate([tile, pad], axis=1)
            r = jnp.minimum(a, tile)
            a = jnp.maximum(a, tile)
            r2 = jnp.minimum(b, r)
            b = jnp.maximum(b, r)
            c = jnp.maximum(c, r2)
        A_ref[...] = a
        B_ref[...] = b
        C_ref[...] = c

    @pl.when(s == 0)
    def _():
        _merge_block(t2_ref[...])

    @pl.when(s == 1)
    def _():
        _merge_block(t1_ref[...])

    @pl.when(j == last)
    def _side_loss():
        a1 = a1_ref[...]
        a2 = a2_ref[...]
        num = jnp.sum(a1 * a2, axis=1, keepdims=True)
        den = (jnp.sqrt(jnp.sum(a1 * a1, axis=1, keepdims=True)) *
               jnp.sqrt(jnp.sum(a2 * a2, axis=1, keepdims=True)))
        d_m1 = (1.0 + MARGIN - num / den) - 1.0
        a = A_ref[...]
        b = B_ref[...]
        c = C_ref[...]
        tot = jnp.zeros((N_ANCHORS, 1), jnp.float32)
        for _ in range(K):
            m = jnp.maximum(
                jnp.max(a, axis=1, keepdims=True),
                jnp.maximum(jnp.max(b, axis=1, keepdims=True),
                            jnp.max(c, axis=1, keepdims=True)))
            tot += jnp.maximum(d_m1 + m, 0.0)
            a = jnp.where(a == m, NEG_FILL, a)
            b = jnp.where(b == m, NEG_FILL, b)
            c = jnp.where(c == m, NEG_FILL, c)
        acc_ref[0, 0] += jnp.sum(tot)

        @pl.when(s == 1)
        def _emit():
            out_ref[...] = jnp.broadcast_to(
                acc_ref[0, 0] / (N_ANCHORS * K), (1, 1))


def _tc_loss(a1, a2, out1, out2):
    return pl.pallas_call(
        _tc_body,
        grid=(2, N_BLOCKS),
        in_specs=[
            pl.BlockSpec((N_ANCHORS, DIM), lambda s, j: (0, 0)),
            pl.BlockSpec((N_ANCHORS, DIM), lambda s, j: (0, 0)),
            pl.BlockSpec((BLOCK_W, DIM), lambda s, j: (j, 0)),
            pl.BlockSpec((BLOCK_W, DIM), lambda s, j: (j, 0)),
        ],
        out_specs=pl.BlockSpec((1, 1), lambda s, j: (0, 0)),
        out_shape=jax.ShapeDtypeStruct((1, 1), jnp.float32),
        scratch_shapes=[
            pltpu.VMEM((N_ANCHORS, DIM), jnp.bfloat16),
            pltpu.VMEM((N_ANCHORS, 128), jnp.float32),
            pltpu.VMEM((N_ANCHORS, 128), jnp.float32),
            pltpu.VMEM((N_ANCHORS, 128), jnp.float32),
            pltpu.SMEM((1, 1), jnp.float32),
        ],
    )(a1, a2, out2, out1)


def kernel(out1, out2, anchor_links):
    anchor1 = anchor_links[:, 0].astype(jnp.int32)
    anchor2 = anchor_links[:, 1].astype(jnp.int32)
    a1, a2 = _sc_gather(anchor1, anchor2, out1, out2)
    loss = _tc_loss(a1, a2, out1, out2)
    return loss[0, 0]


# speedup vs baseline: 568.4533x; 1.1304x over previous
import functools

import jax
import jax.numpy as jnp
from jax import lax
from jax.experimental import pallas as pl
from jax.experimental.pallas import tpu as pltpu
from jax.experimental.pallas import tpu_sc as plsc

N_ANCHORS = 1024
DIM = 128
K = 10
MARGIN = 0.5
NEG_FILL = -3.0
BLOCK_W = 1000
N_BLOCKS = 100000 // BLOCK_W


def _make_sc_gather():
    info = plsc.get_sparse_core_info()
    nc, ns = info.num_cores, info.num_subcores
    nw = nc * ns
    b_per_w = N_ANCHORS // nw

    mesh = plsc.VectorSubcoreMesh(core_axis_name="c", subcore_axis_name="s")

    @functools.partial(
        pl.kernel,
        mesh=mesh,
        out_type=[
            jax.ShapeDtypeStruct((N_ANCHORS, DIM), jnp.float32),
            jax.ShapeDtypeStruct((N_ANCHORS, DIM), jnp.float32),
        ],
        scratch_types=[
            pltpu.VMEM((b_per_w,), jnp.int32),
            pltpu.VMEM((b_per_w,), jnp.int32),
            pltpu.VMEM((b_per_w, DIM), jnp.float32),
            pltpu.VMEM((b_per_w, DIM), jnp.float32),
            pltpu.SemaphoreType.DMA,
            pltpu.SemaphoreType.DMA,
        ],
    )
    def sc_gather(idx1_hbm, idx2_hbm, t1_hbm, t2_hbm, o1_hbm, o2_hbm,
                  idx1_v, idx2_v, rows1_v, rows2_v, sem1, sem2):
        wid = lax.axis_index("s") * nc + lax.axis_index("c")
        base = wid * b_per_w
        pltpu.sync_copy(idx1_hbm.at[pl.ds(base, b_per_w)], idx1_v)
        pltpu.sync_copy(idx2_hbm.at[pl.ds(base, b_per_w)], idx2_v)
        cp1 = pltpu.async_copy(t1_hbm.at[idx1_v], rows1_v, sem1)
        cp2 = pltpu.async_copy(t2_hbm.at[idx2_v], rows2_v, sem2)
        cp1.wait()
        cp2.wait()
        pltpu.sync_copy(rows1_v, o1_hbm.at[pl.ds(base, b_per_w)])
        pltpu.sync_copy(rows2_v, o2_hbm.at[pl.ds(base, b_per_w)])

    return sc_gather


_sc_gather_cache = []


def _sc_gather(anchor1, anchor2, out1, out2):
    if not _sc_gather_cache:
        _sc_gather_cache.append(_make_sc_gather())
    return _sc_gather_cache[0](anchor1, anchor2, out1, out2)


N_TILES = (BLOCK_W + 127) // 128


def _tc_side_body(anc_in_ref, oth_ref, tbl_ref, out_ref, anc_ref,
                  a0_ref, b0_ref, a1_ref, b1_ref):
    j = pl.program_id(0)
    last = N_BLOCKS - 1

    @pl.when(j == 0)
    def _init():
        a = anc_in_ref[...]
        nrm = jnp.maximum(
            jnp.sqrt(jnp.sum(a * a, axis=1, keepdims=True)), 1e-12)
        anc_ref[...] = (a / nrm).astype(jnp.bfloat16)
        fill = jnp.full((N_ANCHORS, 128), NEG_FILL, jnp.float32)
        a0_ref[...] = fill
        b0_ref[...] = fill
        a1_ref[...] = fill
        b1_ref[...] = fill

    blk = tbl_ref[...]
    nrm = jnp.maximum(
        jnp.sqrt(jnp.sum(blk * blk, axis=1, keepdims=True)), 1e-12)
    blkn = (blk / nrm).astype(jnp.bfloat16)
    sims = lax.dot_general(
        anc_ref[...], blkn, (((1,), (1,)), ((), ())),
        preferred_element_type=jnp.float32)
    pa = [a0_ref[...], a1_ref[...]]
    pb = [b0_ref[...], b1_ref[...]]
    for t in range(N_TILES):
        lo = t * 128
        hi = min(lo + 128, BLOCK_W)
        tile = sims[:, lo:hi]
        if hi - lo < 128:
            pad = jnp.full((N_ANCHORS, 128 - (hi - lo)), NEG_FILL,
                           jnp.float32)
            tile = jnp.concatenate([tile, pad], axis=1)
        p = t & 1
        r = jnp.minimum(pa[p], tile)
        pa[p] = jnp.maximum(pa[p], tile)
        pb[p] = jnp.maximum(pb[p], r)
    a0_ref[...] = pa[0]
    a1_ref[...] = pa[1]
    b0_ref[...] = pb[0]
    b1_ref[...] = pb[1]

    @pl.when(j == last)
    def _side_loss():
        x1 = anc_in_ref[...]
        x2 = oth_ref[...]
        num = jnp.sum(x1 * x2, axis=1, keepdims=True)
        den = (jnp.sqrt(jnp.sum(x1 * x1, axis=1, keepdims=True)) *
               jnp.sqrt(jnp.sum(x2 * x2, axis=1, keepdims=True)))
        d_m1 = (1.0 + MARGIN - num / den) - 1.0
        cands = [a0_ref[...], a1_ref[...], b0_ref[...], b1_ref[...]]
        tot = jnp.zeros((N_ANCHORS, 1), jnp.float32)
        for _ in range(K):
            m = None
            for cd in cands:
                mm = jnp.max(cd, axis=1, keepdims=True)
                m = mm if m is None else jnp.maximum(m, mm)
            tot += jnp.maximum(d_m1 + m, 0.0)
            cands = [jnp.where(cd == m, NEG_FILL, cd) for cd in cands]
        out_ref[...] = jnp.broadcast_to(
            jnp.sum(tot) / (N_ANCHORS * K), (1, 1))


def _tc_side_loss(anchors, others, table):
    return pl.pallas_call(
        _tc_side_body,
        grid=(N_BLOCKS,),
        in_specs=[
            pl.BlockSpec((N_ANCHORS, DIM), lambda j: (0, 0)),
            pl.BlockSpec((N_ANCHORS, DIM), lambda j: (0, 0)),
            pl.BlockSpec((BLOCK_W, DIM), lambda j: (j, 0)),
        ],
        out_specs=pl.BlockSpec((1, 1), lambda j: (0, 0)),
        out_shape=jax.ShapeDtypeStruct((1, 1), jnp.float32),
        scratch_shapes=[
            pltpu.VMEM((N_ANCHORS, DIM), jnp.bfloat16),
            pltpu.VMEM((N_ANCHORS, 128), jnp.float32),
            pltpu.VMEM((N_ANCHORS, 128), jnp.float32),
            pltpu.VMEM((N_ANCHORS, 128), jnp.float32),
            pltpu.VMEM((N_ANCHORS, 128), jnp.float32),
        ],
    )(anchors, others, table)


def kernel(out1, out2, anchor_links):
    anchor1 = anchor_links[:, 0].astype(jnp.int32)
    anchor2 = anchor_links[:, 1].astype(jnp.int32)
    a1, a2 = _sc_gather(anchor1, anchor2, out1, out2)
    p1 = _tc_side_loss(a1, a2, out2)
    p2 = _tc_side_loss(a2, a1, out1)
    return p1[0, 0] + p2[0, 0]
